# Initial kernel scaffold; baseline (speedup 1.0000x reference)
import jax
import jax.numpy as jnp
from jax.experimental import pallas as pl


def kernel(input_data, codebooks, W_in, b_in, W_out, b_out, random_vector):
    raise NotImplementedError("write your pallas kernel here")


# speedup vs baseline: 1.5274x; 1.5274x over previous
import functools

import jax
import jax.numpy as jnp
from jax.experimental import pallas as pl

B, DIM, T = 16, 768, 576
K, EDIM = 8192, 256
EPS = 1e-12
KT = 1024


def _nsvq_kernel(x_ref, cb_ref, w_in_ref, b_in_ref, w_out_ref, b_out_ref,
                 rand_ref, out_ref):
    x = x_ref[0]
    enc = jnp.dot(w_in_ref[...], x, preferred_element_type=jnp.float32)
    enc = enc + b_in_ref[...]
    ennorm2 = jnp.sum(enc * enc, axis=0, keepdims=True)

    m = jnp.full((1, T), jnp.inf, dtype=jnp.float32)
    for kt in range(K // KT):
        cb = cb_ref[pl.ds(kt * KT, KT), :]
        cn = jnp.sum(cb * cb, axis=1, keepdims=True)
        s = jnp.dot(cb, enc, preferred_element_type=jnp.float32)
        d = cn - 2.0 * s
        m = jnp.minimum(m, jnp.min(d, axis=0, keepdims=True))

    r = rand_ref[0]
    rnorm = jnp.sqrt(jnp.sum(r * r, axis=0, keepdims=True))
    res = jnp.sqrt(jnp.maximum(ennorm2 + m, 0.0))
    scale = res / (rnorm + EPS)
    q = enc + r * scale
    out = jnp.dot(w_out_ref[...], q, preferred_element_type=jnp.float32)
    out_ref[0] = out + b_out_ref[...]


@functools.partial(jax.jit, static_argnames=())
def kernel(input_data, codebooks, W_in, b_in, W_out, b_out, random_vector):
    rand_t = jnp.transpose(random_vector.reshape(B, T, EDIM), (0, 2, 1))
    b_in2 = b_in.reshape(EDIM, 1)
    b_out2 = b_out.reshape(DIM, 1)

    out = pl.pallas_call(
        _nsvq_kernel,
        grid=(B,),
        in_specs=[
            pl.BlockSpec((1, DIM, T), lambda b: (b, 0, 0)),
            pl.BlockSpec((K, EDIM), lambda b: (0, 0)),
            pl.BlockSpec((EDIM, DIM), lambda b: (0, 0)),
            pl.BlockSpec((EDIM, 1), lambda b: (0, 0)),
            pl.BlockSpec((DIM, EDIM), lambda b: (0, 0)),
            pl.BlockSpec((DIM, 1), lambda b: (0, 0)),
            pl.BlockSpec((1, EDIM, T), lambda b: (b, 0, 0)),
        ],
        out_specs=pl.BlockSpec((1, DIM, T), lambda b: (b, 0, 0)),
        out_shape=jax.ShapeDtypeStruct((B, DIM, T), jnp.float32),
    )(input_data, codebooks, W_in, b_in2, W_out, b_out2, rand_t)
    return out
